# baseline (speedup 1.0000x reference)
import jax
import jax.numpy as jnp
from jax import lax
from jax.experimental import pallas as pl
from jax.experimental.pallas import tpu as pltpu
from jax.experimental.pallas import tpu_sc as plsc

_LANES = 16
_NUM_WORKERS = 16
_ROWS_PER_WORKER = 2


def _make_kernel(H, W, F):
    assert H == _NUM_WORKERS * _ROWS_PER_WORKER and W == H
    assert F % _LANES == 0
    nv = F // _LANES

    mesh = plsc.VectorSubcoreMesh(
        core_axis_name="c", subcore_axis_name="s", num_cores=1, num_subcores=16
    )

    def body(
        row_hbm, col_hbm, out_hbm,
        col_v0, row_v0, col_v1, row_v1,
        cb_v0, rb_v0, cb_v1, rb_v1,
        sem_c0, sem_r0, sem_c1, sem_r1, sem_out,
    ):
        sid = lax.axis_index("s")
        i0 = sid * _ROWS_PER_WORKER
        i1 = i0 + 1

        copies = [
            pltpu.async_copy(col_hbm.at[i0], col_v0, sem_c0),
            pltpu.async_copy(row_hbm.at[i0], row_v0, sem_r0),
            pltpu.async_copy(col_hbm.at[i1], col_v1, sem_c1),
            pltpu.async_copy(row_hbm.at[i1], row_v1, sem_r1),
        ]

        def make_splat(src_v, dst_v):
            regs = [src_v[pl.ds(v * _LANES, _LANES)] for v in range(nv)]

            def splat(j, _):
                for v in range(nv):
                    dst_v[j, pl.ds(v * _LANES, _LANES)] = regs[v]
                return _

            return splat

        half = H // 2
        writes = []
        for copy, src_v, bc_v, dst in (
            (copies[0], col_v0, cb_v0, out_hbm.at[0, :, i0, pl.ds(0, F)]),
            (copies[1], row_v0, rb_v0, out_hbm.at[0, i0, :, pl.ds(F, F)]),
            (copies[2], col_v1, cb_v1, out_hbm.at[0, :, i1, pl.ds(0, F)]),
            (copies[3], row_v1, rb_v1, out_hbm.at[0, i1, :, pl.ds(F, F)]),
        ):
            copy.wait()
            splat = make_splat(src_v, bc_v)
            lax.fori_loop(0, half, splat, 0)
            writes.append(
                pltpu.async_copy(bc_v.at[pl.ds(0, half)], dst.at[pl.ds(0, half)], sem_out)
            )
            lax.fori_loop(half, H, splat, 0)
            writes.append(
                pltpu.async_copy(bc_v.at[pl.ds(half, half)], dst.at[pl.ds(half, half)], sem_out)
            )
        for w in writes:
            w.wait()

    return pl.kernel(
        body,
        out_type=jax.ShapeDtypeStruct((1, H, W, 2 * F), jnp.float32),
        mesh=mesh,
        scratch_types=[
            pltpu.VMEM((F,), jnp.float32),
            pltpu.VMEM((F,), jnp.float32),
            pltpu.VMEM((F,), jnp.float32),
            pltpu.VMEM((F,), jnp.float32),
            pltpu.VMEM((H, F), jnp.float32),
            pltpu.VMEM((W, F), jnp.float32),
            pltpu.VMEM((H, F), jnp.float32),
            pltpu.VMEM((W, F), jnp.float32),
            pltpu.SemaphoreType.DMA,
            pltpu.SemaphoreType.DMA,
            pltpu.SemaphoreType.DMA,
            pltpu.SemaphoreType.DMA,
            pltpu.SemaphoreType.DMA,
        ],
    )


def kernel(inputs, row_embed, col_embed):
    H = inputs.shape[1]
    W = inputs.shape[2]
    F = row_embed.shape[-1]
    return _make_kernel(H, W, F)(row_embed, col_embed)


# speedup vs baseline: 1.0208x; 1.0208x over previous
import jax
import jax.numpy as jnp
from jax import lax
from jax.experimental import pallas as pl
from jax.experimental.pallas import tpu as pltpu
from jax.experimental.pallas import tpu_sc as plsc

_LANES = 16
_NUM_WORKERS = 16
_ROWS_PER_WORKER = 2


def _make_kernel(H, W, F):
    assert H == _NUM_WORKERS * _ROWS_PER_WORKER and W == H
    assert F % _LANES == 0
    nv = F // _LANES

    mesh = plsc.VectorSubcoreMesh(
        core_axis_name="c", subcore_axis_name="s", num_cores=1, num_subcores=16
    )

    def body(
        row_hbm, col_hbm, out_hbm,
        col_v0, row_v0, col_v1, row_v1,
        cb_v0, rb_v0, cb_v1, rb_v1,
        sem_in, sem_out,
    ):
        sid = lax.axis_index("s")
        i0 = sid * _ROWS_PER_WORKER
        i1 = i0 + 1

        copies = [
            pltpu.async_copy(col_hbm.at[i0], col_v0, sem_in),
            pltpu.async_copy(row_hbm.at[i0], row_v0, sem_in),
            pltpu.async_copy(col_hbm.at[i1], col_v1, sem_in),
            pltpu.async_copy(row_hbm.at[i1], row_v1, sem_in),
        ]
        for c in copies:
            c.wait()

        def make_splat(src_v, dst_v):
            regs = [src_v[pl.ds(v * _LANES, _LANES)] for v in range(nv)]

            def splat(j, _):
                for v in range(nv):
                    dst_v[j, pl.ds(v * _LANES, _LANES)] = regs[v]
                return _

            return splat

        writes = []
        for src_v, bc_v, dst in (
            (col_v0, cb_v0, out_hbm.at[0, :, i0, pl.ds(0, F)]),
            (row_v0, rb_v0, out_hbm.at[0, i0, :, pl.ds(F, F)]),
            (col_v1, cb_v1, out_hbm.at[0, :, i1, pl.ds(0, F)]),
            (row_v1, rb_v1, out_hbm.at[0, i1, :, pl.ds(F, F)]),
        ):
            lax.fori_loop(0, H, make_splat(src_v, bc_v), 0)
            writes.append(pltpu.async_copy(bc_v, dst, sem_out))
        for w in writes:
            w.wait()

    return pl.kernel(
        body,
        out_type=jax.ShapeDtypeStruct((1, H, W, 2 * F), jnp.float32),
        mesh=mesh,
        scratch_types=[
            pltpu.VMEM((F,), jnp.float32),
            pltpu.VMEM((F,), jnp.float32),
            pltpu.VMEM((F,), jnp.float32),
            pltpu.VMEM((F,), jnp.float32),
            pltpu.VMEM((H, F), jnp.float32),
            pltpu.VMEM((W, F), jnp.float32),
            pltpu.VMEM((H, F), jnp.float32),
            pltpu.VMEM((W, F), jnp.float32),
            pltpu.SemaphoreType.DMA,
            pltpu.SemaphoreType.DMA,
        ],
    )


def kernel(inputs, row_embed, col_embed):
    H = inputs.shape[1]
    W = inputs.shape[2]
    F = row_embed.shape[-1]
    return _make_kernel(H, W, F)(row_embed, col_embed)
